# baseline (speedup 1.0000x reference)
import functools

import jax
import jax.numpy as jnp
from jax.experimental import pallas as pl
from jax.experimental.pallas import tpu as pltpu

N = 100000
E = 3200000
IN_CH = 128
OUT_CH = 16
HID = 64

_SELU_SCALE = 1.0507009873554805
_SELU_ALPHA = 1.6732632423543772


def _selu(v):
    return _SELU_SCALE * jnp.where(v > 0, v, _SELU_ALPHA * (jnp.exp(jnp.minimum(v, 0.0)) - 1.0))


def _mlp_body(ctx_ref, sm_ref, ses_ref, ov_ref, mask_ref, ttc_ref,
              w1_ref, b1_ref, w2_ref, b2_ref, pred_ref, x_ref):
    h = jnp.concatenate([ctx_ref[...], sm_ref[...], ses_ref[...], ov_ref[...]], axis=-1)
    t = _selu(jnp.dot(h, w1_ref[...], preferred_element_type=jnp.float32) + b1_ref[...])
    p = _selu(jnp.dot(t, w2_ref[...], preferred_element_type=jnp.float32) + b2_ref[...])
    p = p * mask_ref[...]
    pred_ref[...] = p
    x_ref[...] = p + ttc_ref[...]


def _mlp(ctx, sm, ses, ov, mask_f, ttc, W1, b1, W2, b2):
    B = 2000
    grid = (N // B,)
    row_spec = lambda c: pl.BlockSpec((B, c), lambda i: (i, 0))
    full = lambda a, b: pl.BlockSpec((a, b), lambda i: (0, 0))
    return pl.pallas_call(
        _mlp_body,
        grid=grid,
        in_specs=[row_spec(64), row_spec(32), row_spec(16), row_spec(16),
                  pl.BlockSpec((B, 1), lambda i: (i, 0)), row_spec(OUT_CH),
                  full(IN_CH, HID), full(1, HID), full(HID, OUT_CH), full(1, OUT_CH)],
        out_specs=[row_spec(OUT_CH), row_spec(OUT_CH)],
        out_shape=[jax.ShapeDtypeStruct((N, OUT_CH), jnp.float32),
                   jax.ShapeDtypeStruct((N, OUT_CH), jnp.float32)],
    )(ctx, sm, ses, ov, mask_f, ttc, W1, b1, W2, b2)


def kernel(edge_index, stage_end_scale_out_vec, context, stage_metrics,
           real_nodes_batch, overhead, total_time_cumsum, W1, b1, W2, b2):
    mask_f = real_nodes_batch.astype(jnp.float32)[:, None]
    pred, x = _mlp(context, stage_metrics, stage_end_scale_out_vec, overhead,
                   mask_f, total_time_cumsum, W1, b1.reshape(1, HID), W2,
                   b2.reshape(1, OUT_CH))
    src = edge_index[0]
    dst = edge_index[1]
    msg = jnp.take(x, src, axis=0)
    agg = jax.ops.segment_max(msg, dst, num_segments=N)
    agg = jnp.where(jnp.isfinite(agg), agg, 0.0)
    return (pred, agg)


# speedup vs baseline: 12.7519x; 12.7519x over previous
import functools

import jax
import jax.numpy as jnp
from jax import lax
from jax.experimental import pallas as pl
from jax.experimental.pallas import tpu as pltpu
from jax.experimental.pallas import tpu_sc as plsc

N = 100000
E = 3200000
IN_CH = 128
OUT_CH = 16
HID = 64

_SELU_SCALE = 1.0507009873554805
_SELU_ALPHA = 1.6732632423543772

NC = 2
NS = 16
RANGE = N // NS
EH = E // NC
CHUNK = 2560
NCHUNK = EH // CHUNK
GB = 64
OUT_W = OUT_CH // 2
AGG_WORDS = RANGE * OUT_W + 48
DUMMY_BASE = RANGE * OUT_W


def _selu(v):
    return _SELU_SCALE * jnp.where(v > 0, v, _SELU_ALPHA * (jnp.exp(jnp.minimum(v, 0.0)) - 1.0))


def _mlp_body(ctx_ref, sm_ref, ses_ref, ov_ref, mask_ref, ttc_ref,
              w1_ref, b1_ref, w2_ref, b2_ref, pred_ref, x_ref):
    h = jnp.concatenate([ctx_ref[...], sm_ref[...], ses_ref[...], ov_ref[...]], axis=-1)
    t = _selu(jnp.dot(h, w1_ref[...], preferred_element_type=jnp.float32) + b1_ref[...])
    p = _selu(jnp.dot(t, w2_ref[...], preferred_element_type=jnp.float32) + b2_ref[...])
    p = p * mask_ref[...]
    pred_ref[...] = p
    x_ref[...] = p + ttc_ref[...]


def _mlp(ctx, sm, ses, ov, mask_f, ttc, W1, b1, W2, b2):
    B = 2000
    grid = (N // B,)
    row_spec = lambda c: pl.BlockSpec((B, c), lambda i: (i, 0))
    full = lambda a, b: pl.BlockSpec((a, b), lambda i: (0, 0))
    return pl.pallas_call(
        _mlp_body,
        grid=grid,
        in_specs=[row_spec(64), row_spec(32), row_spec(16), row_spec(16),
                  pl.BlockSpec((B, 1), lambda i: (i, 0)), row_spec(OUT_CH),
                  full(IN_CH, HID), full(1, HID), full(HID, OUT_CH), full(1, OUT_CH)],
        out_specs=[row_spec(OUT_CH), row_spec(OUT_CH)],
        out_shape=[jax.ShapeDtypeStruct((N, OUT_CH), jnp.float32),
                   jax.ShapeDtypeStruct((N, OUT_CH), jnp.float32)],
    )(ctx, sm, ses, ov, mask_f, ttc, W1, b1, W2, b2)


def _segmax_body(x_hbm, src_hbm, dst_hbm, part_hbm,
                 dstc, srcc, mrel, msrc, rows4, aggf, xsh,
                 sem_d0, sem_d1, sem_s0, sem_s1, sem_g0, sem_g1, sem_g2, sem_g3):
    c = lax.axis_index("c")
    s = lax.axis_index("s")
    lo = s * RANGE
    ebase = c * EH
    iota16 = lax.iota(jnp.int32, 16)
    neg_inf = plsc.bitcast(jnp.full((32,), -jnp.inf, jnp.bfloat16), jnp.float32)

    pltpu.sync_copy(x_hbm.at[pl.ds(s * RANGE, RANGE)],
                    xsh.at[pl.ds(s * RANGE, RANGE)])

    def init_i(i, carry):
        aggf[pl.ds(i * 16, 16)] = neg_inf
        return carry
    lax.fori_loop(0, AGG_WORDS // 16, init_i, 0)
    plsc.subcore_barrier()

    def start_edges(k):
        pb = lax.rem(k, 2)
        base = ebase + k * CHUNK
        for par, semd, sems in ((0, sem_d0, sem_s0), (1, sem_d1, sem_s1)):
            @pl.when(pb == par)
            def _():
                pltpu.async_copy(dst_hbm.at[pl.ds(base, CHUNK)],
                                 dstc.at[pl.ds(par * CHUNK, CHUNK)], semd)
                pltpu.async_copy(src_hbm.at[pl.ds(base, CHUNK)],
                                 srcc.at[pl.ds(par * CHUNK, CHUNK)], sems)

    def wait_edges(k):
        pb = lax.rem(k, 2)
        base = ebase + k * CHUNK
        for par, semd, sems in ((0, sem_d0, sem_s0), (1, sem_d1, sem_s1)):
            @pl.when(pb == par)
            def _():
                pltpu.make_async_copy(dst_hbm.at[pl.ds(base, CHUNK)],
                                      dstc.at[pl.ds(par * CHUNK, CHUNK)], semd).wait()
                pltpu.make_async_copy(src_hbm.at[pl.ds(base, CHUNK)],
                                      srcc.at[pl.ds(par * CHUNK, CHUNK)], sems).wait()

    sem_g = (sem_g0, sem_g1, sem_g2, sem_g3)
    MCAP = CHUNK + 64

    def fire_gather(b, mb):
        slot = lax.rem(b, 4)
        for r in range(4):
            @pl.when(slot == r)
            def _():
                pltpu.async_copy(xsh.at[msrc.at[pl.ds(mb * MCAP + b * GB, GB)]],
                                 rows4.at[r], sem_g[r])

    def wait_gather(b, mb):
        slot = lax.rem(b, 4)
        for r in range(4):
            @pl.when(slot == r)
            def _():
                pltpu.make_async_copy(xsh.at[msrc.at[pl.ds(mb * MCAP + b * GB, GB)]],
                                      rows4.at[r], sem_g[r]).wait()

    def do_filter(k):
        pb = lax.rem(k, 2)
        cbase = pb * CHUNK
        mbase = pb * MCAP

        @plsc.parallel_loop(0, CHUNK // 64, 1, unroll=4, carry=jnp.int32(0))
        def filt(g, off):
            i0 = cbase + g * 64
            data = []
            cnts = []
            for t in range(4):
                dv = dstc[pl.ds(i0 + t * 16, 16)]
                sv = srcc[pl.ds(i0 + t * 16, 16)]
                rel = dv - lo
                ok = (rel >= 0) & (rel < RANGE)
                rel16 = rel << 3
                data.append((rel16, sv, ok))
                cnts.append(jnp.sum(ok.astype(jnp.int32)))
            offs = [off + mbase, off + mbase + cnts[0],
                    off + mbase + cnts[0] + cnts[1],
                    off + mbase + cnts[0] + cnts[1] + cnts[2]]
            for t in range(4):
                rel16, sv, ok = data[t]
                plsc.store_compressed(mrel.at[pl.ds(offs[t], 16)], rel16, mask=ok)
                plsc.store_compressed(msrc.at[pl.ds(offs[t], 16)], sv, mask=ok)
            return off + cnts[0] + cnts[1] + cnts[2] + cnts[3]

        m = filt

        dummy_rel = jnp.full((16,), DUMMY_BASE, jnp.int32) + iota16
        zero16 = jnp.zeros((16,), jnp.int32)
        for t in range(4):
            mrel[pl.ds(mbase + m + t * 16, 16)] = dummy_rel
            msrc[pl.ds(mbase + m + t * 16, 16)] = zero16
        return (m + (GB - 1)) // GB

    def drain(k, nb):
        mb = lax.rem(k, 2)

        def upd(b, carry):
            wait_gather(b, mb)
            slot = lax.rem(b, 4)
            for t in range(4):
                relg = mrel[pl.ds(mb * MCAP + b * GB + t * 16, 16)]
                occ, _last = plsc.scan_count(relg)
                kmax = jnp.max(occ)
                m0 = occ == 0
                rbase = iota16 + t * 16

                def one_round(mask_r):
                    slot16 = jnp.full((16,), slot, jnp.int32)
                    cols = [plsc.load_gather(
                        rows4, [slot16, rbase, jnp.full((16,), ch, jnp.int32)])
                        for ch in range(OUT_W)]
                    curs = [plsc.load_gather(aggf, [relg + ch]) for ch in range(OUT_W)]
                    news = [plsc.bitcast(
                        jnp.maximum(plsc.bitcast(cu, jnp.bfloat16),
                                    plsc.bitcast(co, jnp.bfloat16)), jnp.float32)
                        for cu, co in zip(curs, cols)]
                    for ch in range(OUT_W):
                        plsc.store_scatter(aggf, [relg + ch], news[ch], mask=mask_r)

                one_round(m0)

                def slow(j, carry2):
                    one_round(occ == j)
                    return carry2
                lax.fori_loop(1, kmax + 1, slow, 0)

            @pl.when(b + 4 < nb)
            def _():
                fire_gather(b + 4, mb)
            return carry
        lax.fori_loop(0, nb, upd, 0)

    def prime_ring(nb, mb):
        for i in range(4):
            @pl.when(i < nb)
            def _():
                fire_gather(jnp.int32(i), mb)

    start_edges(0)
    start_edges(1)
    wait_edges(0)
    nb0 = do_filter(0)
    prime_ring(nb0, 0)

    def chunk_body(k, nb_cur):
        @pl.when(k + 2 < NCHUNK)
        def _():
            start_edges(k + 2)

        nb_next = lax.cond(
            k + 1 < NCHUNK,
            lambda: _filter_next(k + 1),
            lambda: jnp.int32(0))

        drain(k, nb_cur)

        @pl.when(k + 1 < NCHUNK)
        def _():
            prime_ring(nb_next, lax.rem(k + 1, 2))
        return nb_next

    def _filter_next(k1):
        wait_edges(k1)
        return do_filter(k1)

    lax.fori_loop(0, NCHUNK, chunk_body, nb0)

    pltpu.sync_copy(aggf.at[pl.ds(0, RANGE * OUT_W)],
                    part_hbm.at[pl.ds((c * NS + s) * RANGE * OUT_W, RANGE * OUT_W)])


def _segmax(x, src, dst):
    mesh = plsc.VectorSubcoreMesh(core_axis_name="c", subcore_axis_name="s")
    f = pl.kernel(
        _segmax_body,
        out_type=jax.ShapeDtypeStruct((NC * NS * RANGE * OUT_W,), jnp.float32),
        mesh=mesh,
        compiler_params=pltpu.CompilerParams(
            needs_layout_passes=False, use_tc_tiling_on_sc=False),
        scratch_types=[
            pltpu.VMEM((2 * CHUNK,), jnp.int32),
            pltpu.VMEM((2 * CHUNK,), jnp.int32),
            pltpu.VMEM((2 * (CHUNK + 64),), jnp.int32),
            pltpu.VMEM((2 * (CHUNK + 64),), jnp.int32),
            pltpu.VMEM((4, GB, OUT_W), jnp.float32),
            pltpu.VMEM((AGG_WORDS,), jnp.float32),
            pltpu.VMEM_SHARED((N, OUT_W), jnp.float32),
            pltpu.SemaphoreType.DMA,
            pltpu.SemaphoreType.DMA,
            pltpu.SemaphoreType.DMA,
            pltpu.SemaphoreType.DMA,
            pltpu.SemaphoreType.DMA,
            pltpu.SemaphoreType.DMA,
            pltpu.SemaphoreType.DMA,
            pltpu.SemaphoreType.DMA,
        ],
    )
    return f(x, src, dst)


def _merge_body(a_ref, b_ref, out_ref):
    v = jnp.maximum(a_ref[...], b_ref[...])
    out_ref[...] = jnp.where(v == -jnp.inf, 0.0, v)


def _merge(p0, p1):
    B = 2000
    spec = pl.BlockSpec((B, OUT_CH), lambda i: (i, 0))
    return pl.pallas_call(
        _merge_body,
        grid=(N // B,),
        in_specs=[spec, spec],
        out_specs=spec,
        out_shape=jax.ShapeDtypeStruct((N, OUT_CH), jnp.float32),
    )(p0, p1)


def kernel(edge_index, stage_end_scale_out_vec, context, stage_metrics,
           real_nodes_batch, overhead, total_time_cumsum, W1, b1, W2, b2):
    mask_f = real_nodes_batch.astype(jnp.float32)[:, None]
    pred, x = _mlp(context, stage_metrics, stage_end_scale_out_vec, overhead,
                   mask_f, total_time_cumsum, W1, b1.reshape(1, HID), W2,
                   b2.reshape(1, OUT_CH))
    src = edge_index[0].astype(jnp.int32)
    dst = edge_index[1].astype(jnp.int32)
    xp = lax.bitcast_convert_type(
        x.astype(jnp.bfloat16).reshape(N, OUT_W, 2), jnp.float32)
    part = _segmax(xp, src, dst)
    part = lax.bitcast_convert_type(
        part.reshape(NC, N, OUT_W, 1), jnp.bfloat16).reshape(NC, N, OUT_CH)
    part = part.astype(jnp.float32)
    agg = _merge(part[0], part[1])
    return (pred, agg)
